# baseline (speedup 1.0000x reference)
import functools

import jax
import jax.numpy as jnp
from jax import lax
from jax.experimental import pallas as pl
from jax.experimental.pallas import tpu as pltpu
from jax.experimental.pallas import tpu_sc as plsc

B = 1024
L = 50
H = 128
V = 1000

NC = 2
NS = 16
NW = NC * NS
ROWS_PER_W = B // NW
GROUPS = ROWS_PER_W // 16
CHUNK = ROWS_PER_W * L

MASK_SCORE = -60.0


def _scores_body(last_ref, all_ref, wr_ref, ur_ref, vr_ref, e_ref, out_ref):
    lm = lax.dot_general(
        last_ref[...], wr_ref[...], (((1,), (1,)), ((), ())),
        preferred_element_type=jnp.float32)
    am = lax.dot_general(
        all_ref[...], ur_ref[...], (((1,), (1,)), ((), ())),
        preferred_element_type=jnp.float32)
    lmexp = lax.dot_general(
        e_ref[...], lm, (((1,), (0,)), ((), ())),
        preferred_element_type=jnp.float32)
    z = jnp.tanh(am + lmexp)
    out_ref[...] = lax.dot_general(
        z, vr_ref[...], (((1,), (1,)), ((), ())),
        preferred_element_type=jnp.float32)


def _tc_scores(last_memory, all_flat, Wr, Ur, Vr, bb=128):
    R = bb * L
    expand = jnp.repeat(jnp.eye(bb, dtype=jnp.float32), L, axis=0)
    return pl.pallas_call(
        _scores_body,
        grid=(B // bb,),
        in_specs=[
            pl.BlockSpec((bb, H), lambda i: (i, 0)),
            pl.BlockSpec((R, H), lambda i: (i, 0)),
            pl.BlockSpec((H, H), lambda i: (0, 0)),
            pl.BlockSpec((H, H), lambda i: (0, 0)),
            pl.BlockSpec((1, H), lambda i: (0, 0)),
            pl.BlockSpec((R, bb), lambda i: (0, 0)),
        ],
        out_specs=pl.BlockSpec((R, 1), lambda i: (i, 0)),
        out_shape=jax.ShapeDtypeStruct((B * L, 1), jnp.float32),
    )(last_memory, all_flat, Wr, Ur, Vr, expand)


def _sc_scatter_body(s_hbm, seq_hbm, mask_hbm, out_hbm,
                     s_v, seq_v, mask_v, vals_v, acc_v):
    wid = lax.axis_index("s") * NC + lax.axis_index("c")
    in_base = wid * CHUNK
    out_base = wid * (ROWS_PER_W * V)

    pltpu.sync_copy(s_hbm.at[pl.ds(in_base, CHUNK)], s_v)
    pltpu.sync_copy(seq_hbm.at[pl.ds(in_base, CHUNK)], seq_v)
    pltpu.sync_copy(mask_hbm.at[pl.ds(in_base, CHUNK)], mask_v)

    zeros16 = jnp.zeros((16,), jnp.float32)

    def _zero(i, _):
        acc_v[pl.ds(i * 16, 16)] = zeros16
        return 0

    lax.fori_loop(0, (ROWS_PER_W * V) // 16, _zero, 0)

    lane = lax.iota(jnp.int32, 16)
    for g in range(GROUPS):
        row = lane + g * 16
        lin = row * L
        acc_base = row * V
        denom = zeros16
        for l in range(L):
            m = plsc.load_gather(mask_v, [lin + l])
            sv = plsc.load_gather(s_v, [lin + l])
            val = jnp.exp(jnp.where(m != 0, MASK_SCORE, sv))
            plsc.store_scatter(vals_v, [lin + l], val)
            denom = denom + val
        dinv = 1.0 / denom
        for l in range(L):
            col = plsc.load_gather(seq_v, [lin + l])
            val = plsc.load_gather(vals_v, [lin + l])
            plsc.addupdate_scatter(acc_v, [acc_base + col], val * dinv)

    pltpu.sync_copy(acc_v, out_hbm.at[pl.ds(out_base, ROWS_PER_W * V)])


@functools.cache
def _sc_scatter():
    return pl.kernel(
        _sc_scatter_body,
        out_type=jax.ShapeDtypeStruct((B * V,), jnp.float32),
        mesh=plsc.VectorSubcoreMesh(core_axis_name="c", subcore_axis_name="s",
                                    num_cores=NC, num_subcores=NS),
        compiler_params=pltpu.CompilerParams(needs_layout_passes=False),
        scratch_types=[
            pltpu.VMEM((CHUNK,), jnp.float32),
            pltpu.VMEM((CHUNK,), jnp.int32),
            pltpu.VMEM((CHUNK,), jnp.int32),
            pltpu.VMEM((CHUNK,), jnp.float32),
            pltpu.VMEM((ROWS_PER_W * V,), jnp.float32),
        ],
    )


def kernel(seq_item, last_memory, all_memory, mask, item_matrix, Wr, Ur, Vr):
    scores = _tc_scores(last_memory, all_memory.reshape(B * L, H), Wr, Ur, Vr)
    out_flat = _sc_scatter()(scores.reshape(B * L),
                             seq_item.astype(jnp.int32).reshape(B * L),
                             mask.astype(jnp.int32).reshape(B * L))
    return out_flat.reshape(B, V)


# speedup vs baseline: 1.1710x; 1.1710x over previous
import functools

import jax
import jax.numpy as jnp
from jax import lax
from jax.experimental import pallas as pl
from jax.experimental.pallas import tpu as pltpu
from jax.experimental.pallas import tpu_sc as plsc

B = 1024
L = 50
H = 128
V = 1000

NC = 2
NS = 16
NW = NC * NS
ROWS_PER_W = B // NW
GROUPS = ROWS_PER_W // 16
CHUNK = ROWS_PER_W * L

MASK_SCORE = -60.0

PACK_ROWS = 56
TC_BLOCKS = 8
SCORE_STRIDE = PACK_ROWS * 128


def _scores_body(last_ref, all_ref, wr_ref, ur_ref, vr_ref, e_ref, out_ref):
    lm = lax.dot_general(
        last_ref[...], wr_ref[...], (((1,), (1,)), ((), ())),
        preferred_element_type=jnp.float32)
    am = lax.dot_general(
        all_ref[...], ur_ref[...], (((1,), (1,)), ((), ())),
        preferred_element_type=jnp.float32)
    lmexp = lax.dot_general(
        e_ref[...], lm, (((1,), (0,)), ((), ())),
        preferred_element_type=jnp.float32)
    z = jnp.tanh(am + lmexp)
    s = lax.dot_general(
        z, vr_ref[...], (((1,), (1,)), ((), ())),
        preferred_element_type=jnp.float32)
    rows = s.shape[0] // 128
    s2 = s.reshape(rows, 128)
    out_ref[...] = jnp.concatenate(
        [s2, jnp.zeros((PACK_ROWS - rows, 128), jnp.float32)], axis=0)


def _tc_scores(last_memory, all_flat, Wr, Ur, Vr, bb=128):
    R = bb * L
    expand = jnp.repeat(jnp.eye(bb, dtype=jnp.float32), L, axis=0)
    return pl.pallas_call(
        _scores_body,
        grid=(B // bb,),
        in_specs=[
            pl.BlockSpec((bb, H), lambda i: (i, 0)),
            pl.BlockSpec((R, H), lambda i: (i, 0)),
            pl.BlockSpec((H, H), lambda i: (0, 0)),
            pl.BlockSpec((H, H), lambda i: (0, 0)),
            pl.BlockSpec((1, H), lambda i: (0, 0)),
            pl.BlockSpec((R, bb), lambda i: (0, 0)),
        ],
        out_specs=pl.BlockSpec((PACK_ROWS, 128), lambda i: (i, 0)),
        out_shape=jax.ShapeDtypeStruct((TC_BLOCKS * PACK_ROWS, 128),
                                       jnp.float32),
    )(last_memory, all_flat, Wr, Ur, Vr, expand)


def _sc_scatter_body(s_hbm, seq_hbm, mask_hbm, out_hbm,
                     s_v, seq_v, mask_v, vals_v, acc_v):
    wid = lax.axis_index("s") * NC + lax.axis_index("c")
    in_base = wid * CHUNK
    s_base = (wid // 4) * SCORE_STRIDE + (wid % 4) * CHUNK
    out_base = wid * (ROWS_PER_W * V)

    pltpu.sync_copy(s_hbm.at[pl.ds(s_base, CHUNK)], s_v)
    pltpu.sync_copy(seq_hbm.at[pl.ds(in_base, CHUNK)], seq_v)
    pltpu.sync_copy(mask_hbm.at[pl.ds(in_base, CHUNK)], mask_v)

    zeros16 = jnp.zeros((16,), jnp.float32)

    def _zero(i, _):
        acc_v[pl.ds(i * 16, 16)] = zeros16
        return 0

    lax.fori_loop(0, (ROWS_PER_W * V) // 16, _zero, 0)

    lane = lax.iota(jnp.int32, 16)
    for g in range(GROUPS):
        row = lane + g * 16
        lin = row * L
        acc_base = row * V
        denom = zeros16
        for l in range(L):
            m = plsc.load_gather(mask_v, [lin + l])
            sv = plsc.load_gather(s_v, [lin + l])
            val = jnp.exp(jnp.where(m != 0, MASK_SCORE, sv))
            plsc.store_scatter(vals_v, [lin + l], val)
            denom = denom + val
        dinv = 1.0 / denom
        for l in range(L):
            col = plsc.load_gather(seq_v, [lin + l])
            val = plsc.load_gather(vals_v, [lin + l])
            plsc.addupdate_scatter(acc_v, [acc_base + col], val * dinv)

    pltpu.sync_copy(acc_v, out_hbm.at[pl.ds(out_base, ROWS_PER_W * V)])


@functools.cache
def _sc_scatter():
    return pl.kernel(
        _sc_scatter_body,
        out_type=jax.ShapeDtypeStruct((B * V,), jnp.float32),
        mesh=plsc.VectorSubcoreMesh(core_axis_name="c", subcore_axis_name="s",
                                    num_cores=NC, num_subcores=NS),
        compiler_params=pltpu.CompilerParams(needs_layout_passes=False),
        scratch_types=[
            pltpu.VMEM((CHUNK,), jnp.float32),
            pltpu.VMEM((CHUNK,), jnp.int32),
            pltpu.VMEM((CHUNK,), jnp.int32),
            pltpu.VMEM((CHUNK,), jnp.float32),
            pltpu.VMEM((ROWS_PER_W * V,), jnp.float32),
        ],
    )


def kernel(seq_item, last_memory, all_memory, mask, item_matrix, Wr, Ur, Vr):
    scores = _tc_scores(last_memory, all_memory.reshape(B * L, H), Wr, Ur, Vr)
    out_flat = _sc_scatter()(scores.reshape(TC_BLOCKS * SCORE_STRIDE),
                             seq_item.astype(jnp.int32).reshape(B * L),
                             mask.astype(jnp.int32).reshape(B * L))
    return out_flat.reshape(B, V)


# speedup vs baseline: 1.2711x; 1.0855x over previous
import functools

import jax
import jax.numpy as jnp
from jax import lax
from jax.experimental import pallas as pl
from jax.experimental.pallas import tpu as pltpu
from jax.experimental.pallas import tpu_sc as plsc

B = 1024
L = 50
H = 128
V = 1000

NC = 2
NS = 16
NW = NC * NS
ROWS_PER_W = B // NW
GROUPS = ROWS_PER_W // 16
CHUNK = ROWS_PER_W * L

MASK_SCORE = -60.0

PACK_ROWS = 56
TC_BLOCKS = 8
SCORE_STRIDE = PACK_ROWS * 128


def _scores_body(last_ref, all_ref, wr_ref, ur_ref, vr_ref, e_ref, out_ref):
    lm = lax.dot_general(
        last_ref[...], wr_ref[...], (((1,), (1,)), ((), ())),
        preferred_element_type=jnp.float32)
    bb = all_ref.shape[0]
    am = lax.dot_general(
        all_ref[...].reshape(bb * L, H), ur_ref[...],
        (((1,), (1,)), ((), ())),
        preferred_element_type=jnp.float32)
    lmexp = lax.dot_general(
        e_ref[...], lm, (((1,), (0,)), ((), ())),
        preferred_element_type=jnp.float32)
    z = jnp.tanh(am + lmexp)
    s = lax.dot_general(
        z, vr_ref[...], (((1,), (1,)), ((), ())),
        preferred_element_type=jnp.float32)
    rows = s.shape[0] // 128
    s2 = s.reshape(rows, 128)
    out_ref[...] = jnp.concatenate(
        [s2, jnp.zeros((PACK_ROWS - rows, 128), jnp.float32)], axis=0)


def _tc_scores(last_memory, all_memory, Wr, Ur, Vr, bb=128):
    R = bb * L
    expand = jnp.repeat(jnp.eye(bb, dtype=jnp.float32), L, axis=0)
    return pl.pallas_call(
        _scores_body,
        grid=(B // bb,),
        in_specs=[
            pl.BlockSpec((bb, H), lambda i: (i, 0)),
            pl.BlockSpec((bb, L, H), lambda i: (i, 0, 0)),
            pl.BlockSpec((H, H), lambda i: (0, 0)),
            pl.BlockSpec((H, H), lambda i: (0, 0)),
            pl.BlockSpec((1, H), lambda i: (0, 0)),
            pl.BlockSpec((R, bb), lambda i: (0, 0)),
        ],
        out_specs=pl.BlockSpec((PACK_ROWS, 128), lambda i: (i, 0)),
        out_shape=jax.ShapeDtypeStruct((TC_BLOCKS * PACK_ROWS, 128),
                                       jnp.float32),
    )(last_memory, all_memory, Wr, Ur, Vr, expand)


def _sc_scatter_body(s_hbm, seq_hbm, mask_hbm, out_hbm,
                     s_v, seq_v, mask_v, vals_v, acc_v):
    wid = lax.axis_index("s") * NC + lax.axis_index("c")
    in_base = wid * CHUNK
    s_base = (wid // 4) * SCORE_STRIDE + (wid % 4) * CHUNK
    out_base = wid * (ROWS_PER_W * V)

    pltpu.sync_copy(s_hbm.at[pl.ds(s_base, CHUNK)], s_v)
    pltpu.sync_copy(seq_hbm.at[pl.ds(in_base, CHUNK)], seq_v)
    pltpu.sync_copy(mask_hbm.at[pl.ds(in_base, CHUNK)], mask_v)

    zeros16 = jnp.zeros((16,), jnp.float32)

    def _zero(i, _):
        acc_v[pl.ds(i * 16, 16)] = zeros16
        return 0

    lax.fori_loop(0, (ROWS_PER_W * V) // 16, _zero, 0)

    lane = lax.iota(jnp.int32, 16)
    for g in range(GROUPS):
        row = lane + g * 16
        lin = row * L
        acc_base = row * V
        denom = zeros16
        for l in range(L):
            m = plsc.load_gather(mask_v, [lin + l])
            sv = plsc.load_gather(s_v, [lin + l])
            val = jnp.exp(jnp.where(m != 0, MASK_SCORE, sv))
            plsc.store_scatter(vals_v, [lin + l], val)
            denom = denom + val
        dinv = 1.0 / denom
        for l in range(L):
            col = plsc.load_gather(seq_v, [lin + l])
            val = plsc.load_gather(vals_v, [lin + l])
            plsc.addupdate_scatter(acc_v, [acc_base + col], val * dinv)

    pltpu.sync_copy(acc_v, out_hbm.at[pl.ds(out_base, ROWS_PER_W * V)])


@functools.cache
def _sc_scatter():
    return pl.kernel(
        _sc_scatter_body,
        out_type=jax.ShapeDtypeStruct((B * V,), jnp.float32),
        mesh=plsc.VectorSubcoreMesh(core_axis_name="c", subcore_axis_name="s",
                                    num_cores=NC, num_subcores=NS),
        compiler_params=pltpu.CompilerParams(needs_layout_passes=False),
        scratch_types=[
            pltpu.VMEM((CHUNK,), jnp.float32),
            pltpu.VMEM((CHUNK,), jnp.int32),
            pltpu.VMEM((CHUNK,), jnp.int32),
            pltpu.VMEM((CHUNK,), jnp.float32),
            pltpu.VMEM((ROWS_PER_W * V,), jnp.float32),
        ],
    )


def kernel(seq_item, last_memory, all_memory, mask, item_matrix, Wr, Ur, Vr):
    scores = _tc_scores(last_memory, all_memory, Wr, Ur, Vr)
    out_flat = _sc_scatter()(scores.reshape(TC_BLOCKS * SCORE_STRIDE),
                             seq_item.astype(jnp.int32).reshape(B * L),
                             mask.astype(jnp.int32).reshape(B * L))
    return out_flat.reshape(B, V)


# speedup vs baseline: 2.0947x; 1.6480x over previous
import functools

import jax
import jax.numpy as jnp
from jax import lax
from jax.experimental import pallas as pl
from jax.experimental.pallas import tpu as pltpu
from jax.experimental.pallas import tpu_sc as plsc

B = 1024
L = 50
H = 128
V = 1000

NC = 2
NS = 16
NW = NC * NS
ROWS_PER_W = B // NW
GROUPS = ROWS_PER_W // 16

BB = 128
PACK_ROWS = 56
TC_BLOCKS = B // BB

MASK_SCORE = -60.0


def _scores_body(last_ref, all_ref, wr_ref, ur_ref, vr_ref, out_ref):
    lm = lax.dot_general(
        last_ref[...], wr_ref[...], (((1,), (1,)), ((), ())),
        preferred_element_type=jnp.float32)
    am = lax.dot_general(
        all_ref[...].reshape(L * BB, H), ur_ref[...],
        (((1,), (1,)), ((), ())),
        preferred_element_type=jnp.float32)
    z = jnp.tanh(am.reshape(L, BB, H) + lm[None, :, :])
    s = lax.dot_general(
        z.reshape(L * BB, H), vr_ref[...], (((1,), (1,)), ((), ())),
        preferred_element_type=jnp.float32)
    s2 = s.reshape(L, BB)
    out_ref[...] = jnp.concatenate(
        [s2, jnp.zeros((PACK_ROWS - L, BB), jnp.float32)], axis=0)


def _tc_scores(last_memory, all_t, Wr, Ur, Vr):
    return pl.pallas_call(
        _scores_body,
        grid=(TC_BLOCKS,),
        in_specs=[
            pl.BlockSpec((BB, H), lambda i: (i, 0)),
            pl.BlockSpec((L, BB, H), lambda i: (0, i, 0)),
            pl.BlockSpec((H, H), lambda i: (0, 0)),
            pl.BlockSpec((H, H), lambda i: (0, 0)),
            pl.BlockSpec((1, H), lambda i: (0, 0)),
        ],
        out_specs=pl.BlockSpec((PACK_ROWS, BB), lambda i: (i, 0)),
        out_shape=jax.ShapeDtypeStruct((TC_BLOCKS * PACK_ROWS, BB),
                                       jnp.float32),
    )(last_memory, all_t, Wr, Ur, Vr)


def _sc_scatter_body(s_hbm, seq_hbm, mask_hbm, out_hbm,
                     s_v, seq_v, mask_v, vals_v, acc_v):
    wid = lax.axis_index("s") * NC + lax.axis_index("c")
    blk = wid // 4
    sub = wid % 4
    in_base = wid * (ROWS_PER_W * L)
    out_base = wid * (ROWS_PER_W * V)

    pltpu.sync_copy(s_hbm.at[pl.ds(blk * PACK_ROWS, PACK_ROWS), :], s_v)
    pltpu.sync_copy(seq_hbm.at[pl.ds(in_base, ROWS_PER_W * L)], seq_v)
    pltpu.sync_copy(mask_hbm.at[pl.ds(in_base, ROWS_PER_W * L)], mask_v)

    zeros16 = jnp.zeros((16,), jnp.float32)

    def _zero(i, _):
        acc_v[pl.ds(i * 16, 16)] = zeros16
        return 0

    lax.fori_loop(0, (ROWS_PER_W * V) // 16, _zero, 0)

    lane = lax.iota(jnp.int32, 16)
    for g in range(GROUPS):
        acc_base = (lane + g * 16) * V
        lin = (lane + g * 16) * L
        denom = zeros16
        for l in range(L):
            m = plsc.load_gather(mask_v, [lin + l])
            sv = s_v[l, pl.ds(sub * ROWS_PER_W + g * 16, 16)]
            val = jnp.exp(jnp.where(m != 0, MASK_SCORE, sv))
            vals_v[l, pl.ds(g * 16, 16)] = val
            denom = denom + val
        dinv = 1.0 / denom
        for l in range(L):
            col = plsc.load_gather(seq_v, [lin + l])
            val = vals_v[l, pl.ds(g * 16, 16)]
            plsc.addupdate_scatter(acc_v, [acc_base + col], val * dinv)

    pltpu.sync_copy(acc_v, out_hbm.at[pl.ds(out_base, ROWS_PER_W * V)])


@functools.cache
def _sc_scatter():
    return pl.kernel(
        _sc_scatter_body,
        out_type=jax.ShapeDtypeStruct((B * V,), jnp.float32),
        mesh=plsc.VectorSubcoreMesh(core_axis_name="c", subcore_axis_name="s",
                                    num_cores=NC, num_subcores=NS),
        compiler_params=pltpu.CompilerParams(needs_layout_passes=False),
        scratch_types=[
            pltpu.VMEM((PACK_ROWS, BB), jnp.float32),
            pltpu.VMEM((ROWS_PER_W * L,), jnp.int32),
            pltpu.VMEM((ROWS_PER_W * L,), jnp.int32),
            pltpu.VMEM((L, 16 * GROUPS), jnp.float32),
            pltpu.VMEM((ROWS_PER_W * V,), jnp.float32),
        ],
    )


def kernel(seq_item, last_memory, all_memory, mask, item_matrix, Wr, Ur, Vr):
    all_t = jnp.transpose(all_memory, (1, 0, 2))
    scores = _tc_scores(last_memory, all_t, Wr, Ur, Vr)
    out_flat = _sc_scatter()(scores,
                             seq_item.astype(jnp.int32).reshape(B * L),
                             mask.astype(jnp.int32).reshape(B * L))
    return out_flat.reshape(B, V)


# speedup vs baseline: 2.4042x; 1.1478x over previous
import functools

import jax
import jax.numpy as jnp
from jax import lax
from jax.experimental import pallas as pl
from jax.experimental.pallas import tpu as pltpu
from jax.experimental.pallas import tpu_sc as plsc

B = 1024
L = 50
H = 128
V = 1000

NC = 2
NS = 16
NW = NC * NS
ROWS_PER_W = B // NW
GROUPS = ROWS_PER_W // 16

BB = 128
PACK_ROWS = 56
TC_BLOCKS = B // BB

MASK_SCORE = -60.0


def _scores_body(last_ref, all_ref, wr_ref, ur_ref, vr_ref, out_ref):
    lm = lax.dot_general(
        last_ref[...], wr_ref[...], (((1,), (1,)), ((), ())),
        preferred_element_type=jnp.float32)
    am = lax.dot_general(
        all_ref[...].reshape(L * BB, H), ur_ref[...],
        (((1,), (1,)), ((), ())),
        preferred_element_type=jnp.float32)
    z = jnp.tanh(am.reshape(L, BB, H) + lm[None, :, :])
    s = lax.dot_general(
        z.reshape(L * BB, H), vr_ref[...], (((1,), (1,)), ((), ())),
        preferred_element_type=jnp.float32)
    s2 = s.reshape(L, BB)
    out_ref[...] = jnp.concatenate(
        [s2, jnp.zeros((PACK_ROWS - L, BB), jnp.float32)], axis=0)


def _tc_scores(last_memory, all_t, Wr, Ur, Vr):
    return pl.pallas_call(
        _scores_body,
        grid=(TC_BLOCKS,),
        in_specs=[
            pl.BlockSpec((BB, H), lambda i: (i, 0)),
            pl.BlockSpec((L, BB, H), lambda i: (0, i, 0)),
            pl.BlockSpec((H, H), lambda i: (0, 0)),
            pl.BlockSpec((H, H), lambda i: (0, 0)),
            pl.BlockSpec((1, H), lambda i: (0, 0)),
        ],
        out_specs=pl.BlockSpec((PACK_ROWS, BB), lambda i: (i, 0)),
        out_shape=jax.ShapeDtypeStruct((TC_BLOCKS * PACK_ROWS, BB),
                                       jnp.float32),
    )(last_memory, all_t, Wr, Ur, Vr)


def _sc_scatter_body(s_hbm, seq_hbm, mask_hbm, out_hbm,
                     s_v, seq_v, mask_v, vals_v, acc_v):
    wid = lax.axis_index("s") * NC + lax.axis_index("c")
    blk = wid // 4
    sub = wid % 4
    in_base = wid * (ROWS_PER_W * L)
    out_base = wid * (ROWS_PER_W * V)

    pltpu.sync_copy(s_hbm.at[pl.ds(blk * PACK_ROWS, PACK_ROWS), :], s_v)
    pltpu.sync_copy(seq_hbm.at[pl.ds(in_base, ROWS_PER_W * L)], seq_v)
    pltpu.sync_copy(mask_hbm.at[pl.ds(in_base, ROWS_PER_W * L)], mask_v)

    zeros16 = jnp.zeros((16,), jnp.float32)

    def _zero(i, _):
        for u in range(16):
            acc_v[pl.ds(i * 256 + u * 16, 16)] = zeros16
        return 0

    lax.fori_loop(0, (ROWS_PER_W * V) // 256, _zero, 0)

    lane = lax.iota(jnp.int32, 16)
    for g in range(GROUPS):
        acc_base = (lane + g * 16) * V
        lin = (lane + g * 16) * L
        denom = zeros16
        for l in range(L):
            m = plsc.load_gather(mask_v, [lin + l])
            sv = s_v[l, pl.ds(sub * ROWS_PER_W + g * 16, 16)]
            val = jnp.exp(jnp.where(m != 0, MASK_SCORE, sv))
            vals_v[l, pl.ds(g * 16, 16)] = val
            denom = denom + val
        dinv = 1.0 / denom
        for l in range(L):
            col = plsc.load_gather(seq_v, [lin + l])
            val = vals_v[l, pl.ds(g * 16, 16)]
            plsc.addupdate_scatter(acc_v, [acc_base + col], val * dinv)

    pltpu.sync_copy(acc_v, out_hbm.at[pl.ds(out_base, ROWS_PER_W * V)])


@functools.cache
def _sc_scatter():
    return pl.kernel(
        _sc_scatter_body,
        out_type=jax.ShapeDtypeStruct((B * V,), jnp.float32),
        mesh=plsc.VectorSubcoreMesh(core_axis_name="c", subcore_axis_name="s",
                                    num_cores=NC, num_subcores=NS),
        compiler_params=pltpu.CompilerParams(needs_layout_passes=False),
        scratch_types=[
            pltpu.VMEM((PACK_ROWS, BB), jnp.float32),
            pltpu.VMEM((ROWS_PER_W * L,), jnp.int32),
            pltpu.VMEM((ROWS_PER_W * L,), jnp.int32),
            pltpu.VMEM((L, 16 * GROUPS), jnp.float32),
            pltpu.VMEM((ROWS_PER_W * V,), jnp.float32),
        ],
    )


def kernel(seq_item, last_memory, all_memory, mask, item_matrix, Wr, Ur, Vr):
    all_t = jnp.transpose(all_memory, (1, 0, 2))
    scores = _tc_scores(last_memory, all_t, Wr, Ur, Vr)
    out_flat = _sc_scatter()(scores,
                             seq_item.astype(jnp.int32).reshape(B * L),
                             mask.astype(jnp.int32).reshape(B * L))
    return out_flat.reshape(B, V)
